# Initial kernel scaffold; baseline (speedup 1.0000x reference)
import jax
import jax.numpy as jnp
from jax.experimental import pallas as pl


def kernel(pc, key):
    raise NotImplementedError("write your pallas kernel here")


# speedup vs baseline: 5.9414x; 5.9414x over previous
import functools

import jax
import jax.numpy as jnp
from jax import lax
from jax.experimental import pallas as pl
from jax.experimental.pallas import tpu as pltpu
from jax.experimental.pallas import tpu_sc as plsc

G = 1024
M = 64
N = 16384
QB = 128
BIG_I32 = 2**30


def _fps_body(start_ref, px_ref, py_ref, pz_ref, cx_ref, cy_ref, cz_ref):
    x = px_ref[:]
    y = py_ref[:]
    z = pz_ref[:]
    iota = (lax.broadcasted_iota(jnp.int32, (128, 128), 0) * 128
            + lax.broadcasted_iota(jnp.int32, (128, 128), 1))
    iota8 = (lax.broadcasted_iota(jnp.int32, (8, 128), 0) * 128
             + lax.broadcasted_iota(jnp.int32, (8, 128), 1))
    start = start_ref[0]

    m0 = iota == start
    zero = jnp.zeros((), jnp.float32)
    lx = jnp.sum(jnp.where(m0, x, 0.0))
    ly = jnp.sum(jnp.where(m0, y, 0.0))
    lz = jnp.sum(jnp.where(m0, z, 0.0))
    cx = jnp.where(iota8 == 0, lx, zero)
    cy = jnp.where(iota8 == 0, ly, zero)
    cz = jnp.where(iota8 == 0, lz, zero)
    dists = jnp.full((128, 128), jnp.inf, jnp.float32)

    def body(i, carry):
        dists, lx, ly, lz, cx, cy, cz = carry
        dx = x - lx
        dy = y - ly
        dz = z - lz
        d = (dx * dx + dy * dy) + dz * dz
        dists = jnp.minimum(dists, d)
        mx = jnp.max(dists)
        nxt = jnp.min(jnp.where(dists == mx, iota, BIG_I32))
        m = iota == nxt
        lx = jnp.sum(jnp.where(m, x, 0.0))
        ly = jnp.sum(jnp.where(m, y, 0.0))
        lz = jnp.sum(jnp.where(m, z, 0.0))
        sel = iota8 == i
        cx = jnp.where(sel, lx, cx)
        cy = jnp.where(sel, ly, cy)
        cz = jnp.where(sel, lz, cz)
        return dists, lx, ly, lz, cx, cy, cz

    carry = (dists, lx, ly, lz, cx, cy, cz)
    carry = lax.fori_loop(1, G, body, carry)
    _, _, _, _, cx, cy, cz = carry
    cx_ref[:] = cx
    cy_ref[:] = cy
    cz_ref[:] = cz


def _fps(px, py, pz, start):
    out = jax.ShapeDtypeStruct((8, 128), jnp.float32)
    return pl.pallas_call(
        _fps_body,
        out_shape=(out, out, out),
        in_specs=[
            pl.BlockSpec(memory_space=pltpu.MemorySpace.SMEM),
            pl.BlockSpec(memory_space=pltpu.MemorySpace.VMEM),
            pl.BlockSpec(memory_space=pltpu.MemorySpace.VMEM),
            pl.BlockSpec(memory_space=pltpu.MemorySpace.VMEM),
        ],
    )(start, px, py, pz)


def _topk_body(cx_ref, cy_ref, cz_ref, px_ref, py_ref, pz_ref, idx_ref,
               cxe_ref, cye_ref, cze_ref):
    px = px_ref[:]
    py = py_ref[:]
    pz = pz_ref[:]
    qx = cx_ref[:]
    qy = cy_ref[:]
    qz = cz_ref[:]
    ones = jnp.ones((1, M), jnp.float32)
    cxe_ref[:] = qx * ones
    cye_ref[:] = qy * ones
    cze_ref[:] = qz * ones

    def bf(v):
        return v.astype(jnp.bfloat16).astype(jnp.float32)

    mm = (bf(qx) * bf(px) + bf(qy) * bf(py)) + bf(qz) * bf(pz)
    sumq2 = (qx * qx + qy * qy) + qz * qz
    sump2 = (px * px + py * py) + pz * pz
    d = (sumq2 - 2.0 * mm) + sump2

    iota_n = lax.broadcasted_iota(jnp.int32, (QB, N), 1)
    lane64 = lax.broadcasted_iota(jnp.int32, (QB, M), 1)
    inf = jnp.float32(jnp.inf)

    def body(i, carry):
        lastv, lasti, out = carry
        keep = (d > lastv) | ((d == lastv) & (iota_n > lasti))
        dm = jnp.where(keep, d, inf)
        mn = jnp.min(dm, axis=1, keepdims=True)
        nxt = jnp.min(jnp.where(dm == mn, iota_n, BIG_I32), axis=1,
                      keepdims=True)
        out = jnp.where(lane64 == i, nxt, out)
        return mn, nxt, out

    lastv = jnp.full((QB, 1), -jnp.inf, jnp.float32)
    lasti = jnp.full((QB, 1), -1, jnp.int32)
    out = jnp.zeros((QB, M), jnp.int32)
    _, _, out = lax.fori_loop(0, M, body, (lastv, lasti, out))
    idx_ref[:] = out


def _topk(cxc, cyc, czc, px1, py1, pz1):
    return pl.pallas_call(
        _topk_body,
        grid=(G // QB,),
        out_shape=(
            jax.ShapeDtypeStruct((G, M), jnp.int32),
            jax.ShapeDtypeStruct((G, M), jnp.float32),
            jax.ShapeDtypeStruct((G, M), jnp.float32),
            jax.ShapeDtypeStruct((G, M), jnp.float32),
        ),
        in_specs=[
            pl.BlockSpec((QB, 1), lambda b: (b, 0)),
            pl.BlockSpec((QB, 1), lambda b: (b, 0)),
            pl.BlockSpec((QB, 1), lambda b: (b, 0)),
            pl.BlockSpec((1, N), lambda b: (0, 0)),
            pl.BlockSpec((1, N), lambda b: (0, 0)),
            pl.BlockSpec((1, N), lambda b: (0, 0)),
        ],
        out_specs=tuple(
            pl.BlockSpec((QB, M), lambda b: (b, 0)) for _ in range(4)),
    )(cxc, cyc, czc, px1, py1, pz1)


def _sc_gather(px, py, pz, idx, cxe, cye, cze):
    info = plsc.get_sparse_core_info()
    nc, ns, nl = info.num_cores, info.num_subcores, info.num_lanes
    nw = nc * ns
    b = G * M
    b_per_w = b // nw
    mesh = plsc.VectorSubcoreMesh(core_axis_name="c", subcore_axis_name="s")
    o = jax.ShapeDtypeStruct((b,), jnp.float32)

    @functools.partial(
        pl.kernel,
        mesh=mesh,
        compiler_params=pltpu.CompilerParams(needs_layout_passes=False),
        out_type=(o, o, o),
        scratch_types=[
            pltpu.VMEM((N,), jnp.float32),
            pltpu.VMEM((N,), jnp.float32),
            pltpu.VMEM((N,), jnp.float32),
            pltpu.VMEM((b_per_w,), jnp.int32),
            pltpu.VMEM((b_per_w,), jnp.float32),
            pltpu.VMEM((b_per_w,), jnp.float32),
            pltpu.VMEM((b_per_w,), jnp.float32),
            pltpu.VMEM((b_per_w,), jnp.float32),
            pltpu.VMEM((b_per_w,), jnp.float32),
            pltpu.VMEM((b_per_w,), jnp.float32),
        ],
    )
    def k(px_hbm, py_hbm, pz_hbm, idx_hbm, cxe_hbm, cye_hbm, cze_hbm,
          ox_hbm, oy_hbm, oz_hbm,
          x_v, y_v, z_v, idx_v, cx_v, cy_v, cz_v, ox_v, oy_v, oz_v):
        wid = lax.axis_index("s") * nc + lax.axis_index("c")
        base = wid * b_per_w
        pltpu.sync_copy(px_hbm, x_v)
        pltpu.sync_copy(py_hbm, y_v)
        pltpu.sync_copy(pz_hbm, z_v)
        pltpu.sync_copy(idx_hbm.at[pl.ds(base, b_per_w)], idx_v)
        pltpu.sync_copy(cxe_hbm.at[pl.ds(base, b_per_w)], cx_v)
        pltpu.sync_copy(cye_hbm.at[pl.ds(base, b_per_w)], cy_v)
        pltpu.sync_copy(cze_hbm.at[pl.ds(base, b_per_w)], cz_v)

        def body(i, _):
            s = pl.ds(i * nl, nl)
            iv = idx_v[s]
            ox_v[s] = plsc.load_gather(x_v, [iv]) - cx_v[s]
            oy_v[s] = plsc.load_gather(y_v, [iv]) - cy_v[s]
            oz_v[s] = plsc.load_gather(z_v, [iv]) - cz_v[s]
            return 0

        lax.fori_loop(0, b_per_w // nl, body, 0)
        pltpu.sync_copy(ox_v, ox_hbm.at[pl.ds(base, b_per_w)])
        pltpu.sync_copy(oy_v, oy_hbm.at[pl.ds(base, b_per_w)])
        pltpu.sync_copy(oz_v, oz_hbm.at[pl.ds(base, b_per_w)])

    return k(px, py, pz, idx, cxe, cye, cze)


def kernel(pc, key):
    start = jax.random.randint(key, (), 0, N).astype(jnp.int32)
    px = pc[:, 0].reshape(128, 128)
    py = pc[:, 1].reshape(128, 128)
    pz = pc[:, 2].reshape(128, 128)
    cx, cy, cz = _fps(px, py, pz, start.reshape(1))

    idx, cxe, cye, cze = _topk(
        cx.reshape(G, 1), cy.reshape(G, 1), cz.reshape(G, 1),
        pc[:, 0].reshape(1, N), pc[:, 1].reshape(1, N), pc[:, 2].reshape(1, N),
    )

    center = jnp.stack([cx.reshape(G), cy.reshape(G), cz.reshape(G)], axis=-1)
    ox, oy, oz = _sc_gather(
        pc[:, 0], pc[:, 1], pc[:, 2], idx.reshape(G * M),
        cxe.reshape(G * M), cye.reshape(G * M), cze.reshape(G * M))
    neighborhood = jnp.stack([ox, oy, oz], axis=-1).reshape(G, M, 3)
    return (neighborhood, center)
